# initial kernel scaffold (unmeasured)
import jax
import jax.numpy as jnp
from jax import lax
from jax.experimental import pallas as pl
from jax.experimental.pallas import tpu as pltpu

HALF = 2048
D = 2048


def kernel(partial, gamma):
    x_local = partial[0]
    my_y = lax.axis_index("y")
    mine = lax.dynamic_slice(x_local, (my_y * HALF, 0), (HALF, D))
    other = lax.dynamic_slice(x_local, ((1 - my_y) * HALF, 0), (HALF, D))
    mine16 = mine.astype(jnp.bfloat16)
    send16 = other.astype(jnp.bfloat16)
    gamma2 = gamma.reshape(1, D)

    def body(mine_ref, send_ref, gamma_ref, out_ref, recv_ref, send_sem, recv_sem):
        mx = lax.axis_index("x")
        my = lax.axis_index("y")
        mz = lax.axis_index("z")
        nbr = (mx, 1 - my, mz)

        barrier = pltpu.get_barrier_semaphore()
        pl.semaphore_signal(
            barrier, inc=1, device_id=nbr, device_id_type=pl.DeviceIdType.MESH
        )
        pl.semaphore_wait(barrier, 1)

        rdma = pltpu.make_async_remote_copy(
            src_ref=send_ref,
            dst_ref=recv_ref,
            send_sem=send_sem,
            recv_sem=recv_sem,
            device_id=nbr,
            device_id_type=pl.DeviceIdType.MESH,
        )
        rdma.start()
        rdma.wait()

        y = mine_ref[...].astype(jnp.float32) + recv_ref[...].astype(jnp.float32)
        rms = jnp.sqrt(jnp.mean(y * y, axis=-1, keepdims=True) + 1e-6)
        out_ref[...] = y / rms * gamma_ref[...]

    return pl.pallas_call(
        body,
        out_shape=jax.ShapeDtypeStruct((HALF, D), jnp.float32),
        in_specs=[
            pl.BlockSpec(memory_space=pltpu.VMEM),
            pl.BlockSpec(memory_space=pltpu.VMEM),
            pl.BlockSpec(memory_space=pltpu.VMEM),
        ],
        out_specs=pl.BlockSpec(memory_space=pltpu.VMEM),
        scratch_shapes=[
            pltpu.VMEM((HALF, D), jnp.bfloat16),
            pltpu.SemaphoreType.DMA,
            pltpu.SemaphoreType.DMA,
        ],
        compiler_params=pltpu.CompilerParams(collective_id=0),
    )(mine16, send16, gamma2)


# baseline (device time: 141357 ns/iter reference)
import jax
import jax.numpy as jnp
from jax import lax
from jax.experimental import pallas as pl
from jax.experimental.pallas import tpu as pltpu

HALF = 2048
D = 2048


def kernel(partial, gamma):
    x_local = partial[0]
    my_y = lax.axis_index("y")
    mine = lax.dynamic_slice(x_local, (my_y * HALF, 0), (HALF, D))
    other = lax.dynamic_slice(x_local, ((1 - my_y) * HALF, 0), (HALF, D))
    mine16 = mine.astype(jnp.bfloat16)
    send16 = other.astype(jnp.bfloat16)
    gamma2 = gamma.reshape(1, D)

    def body(mine_ref, send_ref, gamma_ref, out_ref, recv_ref, send_sem, recv_sem):
        mx = lax.axis_index("x")
        my = lax.axis_index("y")
        mz = lax.axis_index("z")
        nbr = (mx, 1 - my, mz)

        barrier = pltpu.get_barrier_semaphore()
        pl.semaphore_signal(
            barrier, inc=1, device_id=nbr, device_id_type=pl.DeviceIdType.MESH
        )
        pl.semaphore_wait(barrier, 1)

        rdma = pltpu.make_async_remote_copy(
            src_ref=send_ref,
            dst_ref=recv_ref,
            send_sem=send_sem,
            recv_sem=recv_sem,
            device_id=nbr,
            device_id_type=pl.DeviceIdType.MESH,
        )
        rdma.start()
        rdma.wait()

        y = mine_ref[...].astype(jnp.float32) + recv_ref[...].astype(jnp.float32)
        rms = jnp.sqrt(jnp.mean(y * y, axis=-1, keepdims=True) + 1e-6)
        out_ref[...] = y / rms * gamma_ref[...]

    return pl.pallas_call(
        body,
        out_shape=jax.ShapeDtypeStruct((HALF, D), jnp.float32),
        in_specs=[
            pl.BlockSpec(memory_space=pltpu.VMEM),
            pl.BlockSpec(memory_space=pltpu.VMEM),
            pl.BlockSpec(memory_space=pltpu.VMEM),
        ],
        out_specs=pl.BlockSpec(memory_space=pltpu.VMEM),
        scratch_shapes=[
            pltpu.VMEM((HALF, D), jnp.bfloat16),
            pltpu.SemaphoreType.DMA,
            pltpu.SemaphoreType.DMA,
        ],
        compiler_params=pltpu.CompilerParams(
            collective_id=0, vmem_limit_bytes=100 * 1024 * 1024
        ),
    )(mine16, send16, gamma2)


# device time: 89309 ns/iter; 1.5828x vs baseline; 1.5828x over previous
import jax
import jax.numpy as jnp
from jax import lax
from jax.experimental import pallas as pl
from jax.experimental.pallas import tpu as pltpu

HALF = 2048
D = 2048
QROWS = 512
K = 4
CH = QROWS // K
HCH = CH // 2
N_FLOWS = 5


def kernel(partial, gamma):
    x_local = partial[0]
    my_x = lax.axis_index("x")
    my_y = lax.axis_index("y")
    my_z = lax.axis_index("z")
    myq = 2 * my_x + my_z

    mine16 = lax.dynamic_slice(x_local, (my_y * HALF, 0), (HALF, D)).astype(
        jnp.bfloat16
    )
    send16 = lax.dynamic_slice(
        x_local, ((1 - my_y) * HALF + myq * QROWS, 0), (QROWS, D)
    ).astype(jnp.bfloat16)
    gamma2 = gamma.reshape(1, D)

    def body(mine_ref, send_ref, gamma_ref, out_ref, remote_ref, ssems, rsems):
        mx = lax.axis_index("x")
        my = lax.axis_index("y")
        mz = lax.axis_index("z")
        q_own = 2 * mx + mz
        q_z = 2 * mx + (1 - mz)
        q_x = 2 * (1 - mx) + mz
        q_d = 2 * (1 - mx) + (1 - mz)
        nbr_y = (mx, 1 - my, mz)
        nbr_z = (mx, my, 1 - mz)
        nbr_x = (1 - mx, my, mz)

        barrier = pltpu.get_barrier_semaphore()
        for nbr in (nbr_y, nbr_z, nbr_x):
            pl.semaphore_signal(
                barrier, inc=1, device_id=nbr, device_id_type=pl.DeviceIdType.MESH
            )
        pl.semaphore_wait(barrier, 3)

        def copy(src, dst, flow, c, dev):
            return pltpu.make_async_remote_copy(
                src_ref=src,
                dst_ref=dst,
                send_sem=ssems.at[flow, c],
                recv_sem=rsems.at[flow, c],
                device_id=dev,
                device_id_type=pl.DeviceIdType.MESH,
            )

        y_rdmas = []
        for c in range(K):
            rows = pl.ds(q_own * QROWS + c * CH, CH)
            r = copy(send_ref.at[pl.ds(c * CH, CH)], remote_ref.at[rows], 0, c, nbr_y)
            r.start()
            y_rdmas.append(r)

        def recv_desc(rows, flow, c, dev):
            return copy(remote_ref.at[rows], remote_ref.at[rows], flow, c, dev)

        z_fwds, x_fwds, dx_fwds, dz_fwds = [], [], [], []
        recv1, recv2, recv3, recv4 = [], [], [], []

        for c in range(K):
            rows = pl.ds(q_own * QROWS + c * CH, CH)
            y_rdmas[c].wait_recv()
            rz = copy(remote_ref.at[rows], remote_ref.at[rows], 1, c, nbr_z)
            rz.start()
            z_fwds.append(rz)
            rx = copy(remote_ref.at[rows], remote_ref.at[rows], 2, c, nbr_x)
            rx.start()
            x_fwds.append(rx)

        for c in range(K):
            rows_z = pl.ds(q_z * QROWS + c * CH, CH)
            r1 = recv_desc(rows_z, 1, c, nbr_z)
            r1.wait_recv()
            recv1.append(r1)
            lo = pl.ds(q_z * QROWS + c * CH, HCH)
            rdx = copy(remote_ref.at[lo], remote_ref.at[lo], 3, c, nbr_x)
            rdx.start()
            dx_fwds.append(rdx)

            rows_x = pl.ds(q_x * QROWS + c * CH, CH)
            r2 = recv_desc(rows_x, 2, c, nbr_x)
            r2.wait_recv()
            recv2.append(r2)
            hi = pl.ds(q_x * QROWS + c * CH + HCH, HCH)
            rdz = copy(remote_ref.at[hi], remote_ref.at[hi], 4, c, nbr_z)
            rdz.start()
            dz_fwds.append(rdz)

        for c in range(K):
            lo = pl.ds(q_d * QROWS + c * CH, HCH)
            recv_desc(lo, 3, c, nbr_x).wait_recv()
            hi = pl.ds(q_d * QROWS + c * CH + HCH, HCH)
            recv_desc(hi, 4, c, nbr_z).wait_recv()

        for r in y_rdmas + z_fwds + x_fwds + dx_fwds + dz_fwds:
            r.wait_send()

        y = mine_ref[...].astype(jnp.float32) + remote_ref[...].astype(jnp.float32)
        rms = jnp.sqrt(jnp.mean(y * y, axis=-1, keepdims=True) + 1e-6)
        out_ref[...] = y / rms * gamma_ref[...]

    return pl.pallas_call(
        body,
        out_shape=jax.ShapeDtypeStruct((HALF, D), jnp.float32),
        in_specs=[
            pl.BlockSpec(memory_space=pltpu.VMEM),
            pl.BlockSpec(memory_space=pltpu.VMEM),
            pl.BlockSpec(memory_space=pltpu.VMEM),
        ],
        out_specs=pl.BlockSpec(memory_space=pltpu.VMEM),
        scratch_shapes=[
            pltpu.VMEM((HALF, D), jnp.bfloat16),
            pltpu.SemaphoreType.DMA((N_FLOWS, K)),
            pltpu.SemaphoreType.DMA((N_FLOWS, K)),
        ],
        compiler_params=pltpu.CompilerParams(
            collective_id=0, vmem_limit_bytes=100 * 1024 * 1024
        ),
    )(mine16, send16, gamma2)


# device time: 66144 ns/iter; 2.1371x vs baseline; 1.3502x over previous
import jax
import jax.numpy as jnp
from jax import lax
from jax.experimental import pallas as pl
from jax.experimental.pallas import tpu as pltpu

HALF = 2048
D = 2048
QROWS = 512
K = 8
CH = QROWS // K
HCH = CH // 2
N_FLOWS = 5


def kernel(partial, gamma):
    gamma2 = gamma.reshape(1, D)

    def body(
        partial_ref,
        gamma_ref,
        out_ref,
        mine_ref,
        stage_ref,
        sendq_ref,
        remote_ref,
        ssems,
        rsems,
        dsems,
    ):
        mx = lax.axis_index("x")
        my = lax.axis_index("y")
        mz = lax.axis_index("z")
        q_own = 2 * mx + mz
        q_z = 2 * mx + (1 - mz)
        q_x = 2 * (1 - mx) + mz
        q_d = 2 * (1 - mx) + (1 - mz)
        nbr_y = (mx, 1 - my, mz)
        nbr_z = (mx, my, 1 - mz)
        nbr_x = (1 - mx, my, mz)

        barrier = pltpu.get_barrier_semaphore()
        for nbr in (nbr_y, nbr_z, nbr_x):
            pl.semaphore_signal(
                barrier, inc=1, device_id=nbr, device_id_type=pl.DeviceIdType.MESH
            )
        pl.semaphore_wait(barrier, 3)

        send_base = (1 - my) * HALF + q_own * QROWS
        stage_dmas = []
        for c in range(K):
            d = pltpu.make_async_copy(
                partial_ref.at[0, pl.ds(send_base + c * CH, CH), :],
                stage_ref.at[pl.ds(c * CH, CH)],
                dsems.at[1 + c],
            )
            d.start()
            stage_dmas.append(d)
        mine_dma = pltpu.make_async_copy(
            partial_ref.at[0, pl.ds(my * HALF, HALF), :], mine_ref, dsems.at[0]
        )
        mine_dma.start()

        def copy(src, dst, flow, c, dev):
            return pltpu.make_async_remote_copy(
                src_ref=src,
                dst_ref=dst,
                send_sem=ssems.at[flow, c],
                recv_sem=rsems.at[flow, c],
                device_id=dev,
                device_id_type=pl.DeviceIdType.MESH,
            )

        y_rdmas = []
        for c in range(K):
            ch = pl.ds(c * CH, CH)
            stage_dmas[c].wait()
            sendq_ref[ch, :] = stage_ref[ch, :].astype(jnp.bfloat16)
            r = copy(
                sendq_ref.at[ch],
                remote_ref.at[pl.ds(q_own * QROWS + c * CH, CH)],
                0,
                c,
                nbr_y,
            )
            r.start()
            y_rdmas.append(r)

        def compute_rows(rows):
            yv = mine_ref[rows, :] + remote_ref[rows, :].astype(jnp.float32)
            rms = jnp.sqrt(jnp.mean(yv * yv, axis=-1, keepdims=True) + 1e-6)
            out_ref[rows, :] = yv / rms * gamma_ref[...]

        def recv_desc(rows, flow, c, dev):
            return copy(remote_ref.at[rows], remote_ref.at[rows], flow, c, dev)

        mine_dma.wait()

        z_fwds, x_fwds, dx_fwds, dz_fwds = [], [], [], []
        for c in range(K):
            rows = pl.ds(q_own * QROWS + c * CH, CH)
            y_rdmas[c].wait_recv()
            rz = copy(remote_ref.at[rows], remote_ref.at[rows], 1, c, nbr_z)
            rz.start()
            z_fwds.append(rz)
            rx = copy(remote_ref.at[rows], remote_ref.at[rows], 2, c, nbr_x)
            rx.start()
            x_fwds.append(rx)
            compute_rows(rows)

        for c in range(K):
            rows_z = pl.ds(q_z * QROWS + c * CH, CH)
            recv_desc(rows_z, 1, c, nbr_z).wait_recv()
            lo = pl.ds(q_z * QROWS + c * CH, HCH)
            rdx = copy(remote_ref.at[lo], remote_ref.at[lo], 3, c, nbr_x)
            rdx.start()
            dx_fwds.append(rdx)
            compute_rows(rows_z)

            rows_x = pl.ds(q_x * QROWS + c * CH, CH)
            recv_desc(rows_x, 2, c, nbr_x).wait_recv()
            hi = pl.ds(q_x * QROWS + c * CH + HCH, HCH)
            rdz = copy(remote_ref.at[hi], remote_ref.at[hi], 4, c, nbr_z)
            rdz.start()
            dz_fwds.append(rdz)
            compute_rows(rows_x)

        for c in range(K):
            lo = pl.ds(q_d * QROWS + c * CH, HCH)
            recv_desc(lo, 3, c, nbr_x).wait_recv()
            hi = pl.ds(q_d * QROWS + c * CH + HCH, HCH)
            recv_desc(hi, 4, c, nbr_z).wait_recv()
            compute_rows(pl.ds(q_d * QROWS + c * CH, CH))

        for r in y_rdmas + z_fwds + x_fwds + dx_fwds + dz_fwds:
            r.wait_send()

    return pl.pallas_call(
        body,
        out_shape=jax.ShapeDtypeStruct((HALF, D), jnp.float32),
        in_specs=[
            pl.BlockSpec(memory_space=pl.ANY),
            pl.BlockSpec(memory_space=pltpu.VMEM),
        ],
        out_specs=pl.BlockSpec(memory_space=pltpu.VMEM),
        scratch_shapes=[
            pltpu.VMEM((HALF, D), jnp.float32),
            pltpu.VMEM((QROWS, D), jnp.float32),
            pltpu.VMEM((QROWS, D), jnp.bfloat16),
            pltpu.VMEM((HALF, D), jnp.bfloat16),
            pltpu.SemaphoreType.DMA((N_FLOWS, K)),
            pltpu.SemaphoreType.DMA((N_FLOWS, K)),
            pltpu.SemaphoreType.DMA((1 + K,)),
        ],
        compiler_params=pltpu.CompilerParams(
            collective_id=0, vmem_limit_bytes=100 * 1024 * 1024
        ),
    )(partial, gamma2)


# device time: 61260 ns/iter; 2.3075x vs baseline; 1.0797x over previous
import jax
import jax.numpy as jnp
from jax import lax
from jax.experimental import pallas as pl
from jax.experimental.pallas import tpu as pltpu

HALF = 2048
D = 2048
QROWS = 512
K = 8
CH = QROWS // K
HCH = CH // 2
N_FLOWS = 5


def kernel(partial, gamma):
    gamma2 = gamma.reshape(1, D)

    def body(
        partial_ref,
        gamma_ref,
        out_ref,
        mine_ref,
        stage_ref,
        sendq_ref,
        remote_ref,
        ostage_ref,
        ssems,
        rsems,
        dsems,
        osems,
    ):
        mx = lax.axis_index("x")
        my = lax.axis_index("y")
        mz = lax.axis_index("z")
        q_own = 2 * mx + mz
        q_z = 2 * mx + (1 - mz)
        q_x = 2 * (1 - mx) + mz
        q_d = 2 * (1 - mx) + (1 - mz)
        nbr_y = (mx, 1 - my, mz)
        nbr_z = (mx, my, 1 - mz)
        nbr_x = (1 - mx, my, mz)

        barrier = pltpu.get_barrier_semaphore()
        for nbr in (nbr_y, nbr_z, nbr_x):
            pl.semaphore_signal(
                barrier, inc=1, device_id=nbr, device_id_type=pl.DeviceIdType.MESH
            )
        pl.semaphore_wait(barrier, 3)

        send_base = (1 - my) * HALF + q_own * QROWS
        stage_dmas = []
        for c in range(K):
            d = pltpu.make_async_copy(
                partial_ref.at[0, pl.ds(send_base + c * CH, CH), :],
                stage_ref.at[pl.ds(c * CH, CH)],
                dsems.at[1 + c],
            )
            d.start()
            stage_dmas.append(d)
        mine_dma = pltpu.make_async_copy(
            partial_ref.at[0, pl.ds(my * HALF, HALF), :], mine_ref, dsems.at[0]
        )
        mine_dma.start()

        def copy(src, dst, flow, c, dev):
            return pltpu.make_async_remote_copy(
                src_ref=src,
                dst_ref=dst,
                send_sem=ssems.at[flow, c],
                recv_sem=rsems.at[flow, c],
                device_id=dev,
                device_id_type=pl.DeviceIdType.MESH,
            )

        y_rdmas = []
        for c in range(K):
            ch = pl.ds(c * CH, CH)
            stage_dmas[c].wait()
            sendq_ref[ch, :] = stage_ref[ch, :].astype(jnp.bfloat16)
            r = copy(
                sendq_ref.at[ch],
                remote_ref.at[pl.ds(q_own * QROWS + c * CH, CH)],
                0,
                c,
                nbr_y,
            )
            r.start()
            y_rdmas.append(r)

        def compute_rows(rows):
            yv = mine_ref[rows, :] + remote_ref[rows, :].astype(jnp.float32)
            rms = jnp.sqrt(jnp.mean(yv * yv, axis=-1, keepdims=True) + 1e-6)
            ostage_ref[rows, :] = yv / rms * gamma_ref[...]

        out_dmas = []

        def flush_quarter(qidx, osem):
            rows = pl.ds(qidx * QROWS, QROWS)
            d = pltpu.make_async_copy(ostage_ref.at[rows], out_ref.at[rows], osem)
            d.start()
            out_dmas.append(d)

        def recv_desc(rows, flow, c, dev):
            return copy(remote_ref.at[rows], remote_ref.at[rows], flow, c, dev)

        z_fwds, x_fwds, dx_fwds, dz_fwds = [], [], [], []
        for c in range(K):
            rows = pl.ds(q_own * QROWS + c * CH, CH)
            y_rdmas[c].wait_recv()
            rz = copy(remote_ref.at[rows], remote_ref.at[rows], 1, c, nbr_z)
            rz.start()
            z_fwds.append(rz)
            rx = copy(remote_ref.at[rows], remote_ref.at[rows], 2, c, nbr_x)
            rx.start()
            x_fwds.append(rx)

        mine_dma.wait()
        compute_rows(pl.ds(q_own * QROWS, QROWS))
        flush_quarter(q_own, osems.at[0])

        for c in range(K):
            recv_desc(pl.ds(q_z * QROWS + c * CH, CH), 1, c, nbr_z).wait_recv()
            lo = pl.ds(q_z * QROWS + c * CH, HCH)
            rdx = copy(remote_ref.at[lo], remote_ref.at[lo], 3, c, nbr_x)
            rdx.start()
            dx_fwds.append(rdx)

            recv_desc(pl.ds(q_x * QROWS + c * CH, CH), 2, c, nbr_x).wait_recv()
            hi = pl.ds(q_x * QROWS + c * CH + HCH, HCH)
            rdz = copy(remote_ref.at[hi], remote_ref.at[hi], 4, c, nbr_z)
            rdz.start()
            dz_fwds.append(rdz)

        compute_rows(pl.ds(q_z * QROWS, QROWS))
        flush_quarter(q_z, osems.at[1])
        compute_rows(pl.ds(q_x * QROWS, QROWS))
        flush_quarter(q_x, osems.at[2])

        for c in range(K):
            recv_desc(pl.ds(q_d * QROWS + c * CH, HCH), 3, c, nbr_x).wait_recv()
            recv_desc(pl.ds(q_d * QROWS + c * CH + HCH, HCH), 4, c, nbr_z).wait_recv()
            compute_rows(pl.ds(q_d * QROWS + c * CH, CH))
        flush_quarter(q_d, osems.at[3])

        for r in y_rdmas + z_fwds + x_fwds + dx_fwds + dz_fwds:
            r.wait_send()
        for d in out_dmas:
            d.wait()

    return pl.pallas_call(
        body,
        out_shape=jax.ShapeDtypeStruct((HALF, D), jnp.float32),
        in_specs=[
            pl.BlockSpec(memory_space=pl.ANY),
            pl.BlockSpec(memory_space=pltpu.VMEM),
        ],
        out_specs=pl.BlockSpec(memory_space=pl.ANY),
        scratch_shapes=[
            pltpu.VMEM((HALF, D), jnp.float32),
            pltpu.VMEM((QROWS, D), jnp.float32),
            pltpu.VMEM((QROWS, D), jnp.bfloat16),
            pltpu.VMEM((HALF, D), jnp.bfloat16),
            pltpu.VMEM((HALF, D), jnp.float32),
            pltpu.SemaphoreType.DMA((N_FLOWS, K)),
            pltpu.SemaphoreType.DMA((N_FLOWS, K)),
            pltpu.SemaphoreType.DMA((1 + K,)),
            pltpu.SemaphoreType.DMA((4,)),
        ],
        compiler_params=pltpu.CompilerParams(
            collective_id=0, vmem_limit_bytes=100 * 1024 * 1024
        ),
    )(partial, gamma2)


# device time: 56630 ns/iter; 2.4962x vs baseline; 1.0818x over previous
import jax
import jax.numpy as jnp
from jax import lax
from jax.experimental import pallas as pl
from jax.experimental.pallas import tpu as pltpu

HALF = 2048
D = 2048
QROWS = 512
K = 8
CH = QROWS // K
HCH = CH // 2
N_FLOWS = 5


def kernel(partial, gamma):
    gamma2 = gamma.reshape(1, D)

    def body(
        partial_ref,
        gamma_ref,
        out_ref,
        mine_ref,
        stage_ref,
        sendq_ref,
        remote_ref,
        ostage_ref,
        ssems,
        rsems,
        dsems,
        osems,
    ):
        mx = lax.axis_index("x")
        my = lax.axis_index("y")
        mz = lax.axis_index("z")
        q_own = 2 * mx + mz
        q_z = 2 * mx + (1 - mz)
        q_x = 2 * (1 - mx) + mz
        q_d = 2 * (1 - mx) + (1 - mz)
        nbr_y = (mx, 1 - my, mz)
        nbr_z = (mx, my, 1 - mz)
        nbr_x = (1 - mx, my, mz)

        barrier = pltpu.get_barrier_semaphore()
        for nbr in (nbr_y, nbr_z, nbr_x):
            pl.semaphore_signal(
                barrier, inc=1, device_id=nbr, device_id_type=pl.DeviceIdType.MESH
            )
        pl.semaphore_wait(barrier, 3)

        send_base = (1 - my) * HALF + q_own * QROWS
        stage_dmas = []
        for c in range(K):
            d = pltpu.make_async_copy(
                partial_ref.at[0, pl.ds(send_base + c * CH, CH), :],
                stage_ref.at[pl.ds(c * CH, CH)],
                dsems.at[1 + c],
            )
            d.start()
            stage_dmas.append(d)
        mine_dma = pltpu.make_async_copy(
            partial_ref.at[0, pl.ds(my * HALF, HALF), :], mine_ref, dsems.at[0]
        )
        mine_dma.start()

        def copy(src, dst, flow, c, dev):
            return pltpu.make_async_remote_copy(
                src_ref=src,
                dst_ref=dst,
                send_sem=ssems.at[flow, c],
                recv_sem=rsems.at[flow, c],
                device_id=dev,
                device_id_type=pl.DeviceIdType.MESH,
            )

        y_rdmas = []
        for c in range(K):
            ch = pl.ds(c * CH, CH)
            stage_dmas[c].wait()
            sendq_ref[ch, :] = stage_ref[ch, :].astype(jnp.bfloat16)
            r = copy(
                sendq_ref.at[ch],
                remote_ref.at[pl.ds(q_own * QROWS + c * CH, CH)],
                0,
                c,
                nbr_y,
            )
            r.start()
            y_rdmas.append(r)

        def compute_rows(rows):
            yv = mine_ref[rows, :] + remote_ref[rows, :].astype(jnp.float32)
            rms = jnp.sqrt(jnp.mean(yv * yv, axis=-1, keepdims=True) + 1e-6)
            ostage_ref[rows, :] = (yv / rms * gamma_ref[...]).astype(jnp.bfloat16)

        out_dmas = []

        def flush_quarter(qidx, osem):
            rows = pl.ds(qidx * QROWS, QROWS)
            d = pltpu.make_async_copy(ostage_ref.at[rows], out_ref.at[rows], osem)
            d.start()
            out_dmas.append(d)

        def recv_desc(rows, flow, c, dev):
            return copy(remote_ref.at[rows], remote_ref.at[rows], flow, c, dev)

        mine_dma.wait()
        z_fwds, x_fwds, dx_fwds, dz_fwds = [], [], [], []
        for c in range(K):
            rows = pl.ds(q_own * QROWS + c * CH, CH)
            y_rdmas[c].wait_recv()
            rz = copy(remote_ref.at[rows], remote_ref.at[rows], 1, c, nbr_z)
            rz.start()
            z_fwds.append(rz)
            rx = copy(remote_ref.at[rows], remote_ref.at[rows], 2, c, nbr_x)
            rx.start()
            x_fwds.append(rx)
            compute_rows(rows)
        flush_quarter(q_own, osems.at[0])

        for c in range(K):
            rows_z = pl.ds(q_z * QROWS + c * CH, CH)
            recv_desc(rows_z, 1, c, nbr_z).wait_recv()
            lo = pl.ds(q_z * QROWS + c * CH, HCH)
            rdx = copy(remote_ref.at[lo], remote_ref.at[lo], 3, c, nbr_x)
            rdx.start()
            dx_fwds.append(rdx)

            rows_x = pl.ds(q_x * QROWS + c * CH, CH)
            recv_desc(rows_x, 2, c, nbr_x).wait_recv()
            hi = pl.ds(q_x * QROWS + c * CH + HCH, HCH)
            rdz = copy(remote_ref.at[hi], remote_ref.at[hi], 4, c, nbr_z)
            rdz.start()
            dz_fwds.append(rdz)

            compute_rows(rows_z)
            compute_rows(rows_x)

        flush_quarter(q_z, osems.at[1])
        flush_quarter(q_x, osems.at[2])

        for c in range(K):
            recv_desc(pl.ds(q_d * QROWS + c * CH, HCH), 3, c, nbr_x).wait_recv()
            recv_desc(pl.ds(q_d * QROWS + c * CH + HCH, HCH), 4, c, nbr_z).wait_recv()
            compute_rows(pl.ds(q_d * QROWS + c * CH, CH))
        flush_quarter(q_d, osems.at[3])

        for r in y_rdmas + z_fwds + x_fwds + dx_fwds + dz_fwds:
            r.wait_send()
        for d in out_dmas:
            d.wait()

    return pl.pallas_call(
        body,
        out_shape=jax.ShapeDtypeStruct((HALF, D), jnp.bfloat16),
        in_specs=[
            pl.BlockSpec(memory_space=pl.ANY),
            pl.BlockSpec(memory_space=pltpu.VMEM),
        ],
        out_specs=pl.BlockSpec(memory_space=pl.ANY),
        scratch_shapes=[
            pltpu.VMEM((HALF, D), jnp.float32),
            pltpu.VMEM((QROWS, D), jnp.float32),
            pltpu.VMEM((QROWS, D), jnp.bfloat16),
            pltpu.VMEM((HALF, D), jnp.bfloat16),
            pltpu.VMEM((HALF, D), jnp.bfloat16),
            pltpu.SemaphoreType.DMA((N_FLOWS, K)),
            pltpu.SemaphoreType.DMA((N_FLOWS, K)),
            pltpu.SemaphoreType.DMA((1 + K,)),
            pltpu.SemaphoreType.DMA((4,)),
        ],
        compiler_params=pltpu.CompilerParams(
            collective_id=0, vmem_limit_bytes=100 * 1024 * 1024
        ),
    )(partial, gamma2)


# device time: 54807 ns/iter; 2.5792x vs baseline; 1.0333x over previous
import jax
import jax.numpy as jnp
from jax import lax
from jax.experimental import pallas as pl
from jax.experimental.pallas import tpu as pltpu

HALF = 2048
D = 2048
QROWS = 512
K = 8
CH = QROWS // K
N_FLOWS = 6
YD = 2
X_CHUNKS = (2, 3, 4)
Z_CHUNKS = (5, 6, 7)


def kernel(partial, gamma):
    gamma2 = gamma.reshape(1, D)

    def body(
        partial_ref,
        gamma_ref,
        out_ref,
        mine_ref,
        stage_ref,
        sendq_ref,
        remote_ref,
        ostage_ref,
        ssems,
        rsems,
        dsems,
        osems,
    ):
        mx = lax.axis_index("x")
        my = lax.axis_index("y")
        mz = lax.axis_index("z")
        q_own = 2 * mx + mz
        q_z = 2 * mx + (1 - mz)
        q_x = 2 * (1 - mx) + mz
        q_d = 2 * (1 - mx) + (1 - mz)
        nbr_y = (mx, 1 - my, mz)
        nbr_z = (mx, my, 1 - mz)
        nbr_x = (1 - mx, my, mz)

        barrier = pltpu.get_barrier_semaphore()
        for nbr in (nbr_y, nbr_z, nbr_x):
            pl.semaphore_signal(
                barrier, inc=1, device_id=nbr, device_id_type=pl.DeviceIdType.MESH
            )
        pl.semaphore_wait(barrier, 3)

        other_base = (1 - my) * HALF
        stage_dmas = []
        for c in range(K):
            d = pltpu.make_async_copy(
                partial_ref.at[0, pl.ds(other_base + q_own * QROWS + c * CH, CH), :],
                stage_ref.at[pl.ds(c * CH, CH)],
                dsems.at[1 + c],
            )
            d.start()
            stage_dmas.append(d)
        for c in range(YD):
            d = pltpu.make_async_copy(
                partial_ref.at[0, pl.ds(other_base + q_d * QROWS + c * CH, CH), :],
                stage_ref.at[pl.ds(QROWS + c * CH, CH)],
                dsems.at[1 + K + c],
            )
            d.start()
            stage_dmas.append(d)
        mine_dma = pltpu.make_async_copy(
            partial_ref.at[0, pl.ds(my * HALF, HALF), :], mine_ref, dsems.at[0]
        )
        mine_dma.start()

        def copy(src, dst, flow, c, dev):
            return pltpu.make_async_remote_copy(
                src_ref=src,
                dst_ref=dst,
                send_sem=ssems.at[flow, c],
                recv_sem=rsems.at[flow, c],
                device_id=dev,
                device_id_type=pl.DeviceIdType.MESH,
            )

        y_rdmas = []
        for c in range(K):
            ch = pl.ds(c * CH, CH)
            stage_dmas[c].wait()
            sendq_ref[ch, :] = stage_ref[ch, :].astype(jnp.bfloat16)
            r = copy(
                sendq_ref.at[ch],
                remote_ref.at[pl.ds(q_own * QROWS + c * CH, CH)],
                0,
                c,
                nbr_y,
            )
            r.start()
            y_rdmas.append(r)
        yd_rdmas = []
        for c in range(YD):
            ch = pl.ds(QROWS + c * CH, CH)
            stage_dmas[K + c].wait()
            sendq_ref[ch, :] = stage_ref[ch, :].astype(jnp.bfloat16)
            r = copy(
                sendq_ref.at[ch],
                remote_ref.at[pl.ds(q_d * QROWS + c * CH, CH)],
                5,
                c,
                nbr_y,
            )
            r.start()
            yd_rdmas.append(r)

        def compute_rows(rows):
            yv = mine_ref[rows, :] + remote_ref[rows, :].astype(jnp.float32)
            rms = jnp.sqrt(jnp.mean(yv * yv, axis=-1, keepdims=True) + 1e-6)
            ostage_ref[rows, :] = (yv / rms * gamma_ref[...]).astype(jnp.bfloat16)

        out_dmas = []

        def flush_quarter(qidx, osem):
            rows = pl.ds(qidx * QROWS, QROWS)
            d = pltpu.make_async_copy(ostage_ref.at[rows], out_ref.at[rows], osem)
            d.start()
            out_dmas.append(d)

        def recv_desc(rows, flow, c, dev):
            return copy(remote_ref.at[rows], remote_ref.at[rows], flow, c, dev)

        mine_dma.wait()
        z_fwds, x_fwds, d_fwds = [], [], []
        for c in range(K):
            rows = pl.ds(q_own * QROWS + c * CH, CH)
            y_rdmas[c].wait_recv()
            rz = copy(remote_ref.at[rows], remote_ref.at[rows], 1, c, nbr_z)
            rz.start()
            z_fwds.append(rz)
            rx = copy(remote_ref.at[rows], remote_ref.at[rows], 2, c, nbr_x)
            rx.start()
            x_fwds.append(rx)
            compute_rows(rows)
        flush_quarter(q_own, osems.at[0])

        for c in range(K):
            rows_z = pl.ds(q_z * QROWS + c * CH, CH)
            recv_desc(rows_z, 1, c, nbr_z).wait_recv()
            if c in X_CHUNKS:
                rdx = copy(remote_ref.at[rows_z], remote_ref.at[rows_z], 3, c, nbr_x)
                rdx.start()
                d_fwds.append(rdx)

            rows_x = pl.ds(q_x * QROWS + c * CH, CH)
            recv_desc(rows_x, 2, c, nbr_x).wait_recv()
            if c in Z_CHUNKS:
                rdz = copy(remote_ref.at[rows_x], remote_ref.at[rows_x], 4, c, nbr_z)
                rdz.start()
                d_fwds.append(rdz)

            compute_rows(rows_z)
            compute_rows(rows_x)

        flush_quarter(q_z, osems.at[1])
        flush_quarter(q_x, osems.at[2])

        for c in range(YD):
            yd_rdmas[c].wait_recv()
            compute_rows(pl.ds(q_d * QROWS + c * CH, CH))
        for c in X_CHUNKS:
            rows = pl.ds(q_d * QROWS + c * CH, CH)
            recv_desc(rows, 3, c, nbr_x).wait_recv()
            compute_rows(rows)
        for c in Z_CHUNKS:
            rows = pl.ds(q_d * QROWS + c * CH, CH)
            recv_desc(rows, 4, c, nbr_z).wait_recv()
            compute_rows(rows)
        flush_quarter(q_d, osems.at[3])

        for r in y_rdmas + yd_rdmas + z_fwds + x_fwds + d_fwds:
            r.wait_send()
        for d in out_dmas:
            d.wait()

    return pl.pallas_call(
        body,
        out_shape=jax.ShapeDtypeStruct((HALF, D), jnp.bfloat16),
        in_specs=[
            pl.BlockSpec(memory_space=pl.ANY),
            pl.BlockSpec(memory_space=pltpu.VMEM),
        ],
        out_specs=pl.BlockSpec(memory_space=pl.ANY),
        scratch_shapes=[
            pltpu.VMEM((HALF, D), jnp.float32),
            pltpu.VMEM((QROWS + YD * CH, D), jnp.float32),
            pltpu.VMEM((QROWS + YD * CH, D), jnp.bfloat16),
            pltpu.VMEM((HALF, D), jnp.bfloat16),
            pltpu.VMEM((HALF, D), jnp.bfloat16),
            pltpu.SemaphoreType.DMA((N_FLOWS, K)),
            pltpu.SemaphoreType.DMA((N_FLOWS, K)),
            pltpu.SemaphoreType.DMA((1 + K + YD,)),
            pltpu.SemaphoreType.DMA((4,)),
        ],
        compiler_params=pltpu.CompilerParams(
            collective_id=0, vmem_limit_bytes=100 * 1024 * 1024
        ),
    )(partial, gamma2)
